# baseline (speedup 1.0000x reference)
import functools
import jax
import jax.numpy as jnp
from jax import lax
from jax.experimental import pallas as pl
from jax.experimental.pallas import tpu as pltpu
from jax.experimental.pallas import tpu_sc as plsc

CH = 16
UN = 8


def _make_sc(batch, seq_len, d_model):
    n_workers = 32
    seq_per_w = seq_len // n_workers
    n_chunks = seq_per_w // CH
    rows = batch * seq_len
    mesh = plsc.VectorSubcoreMesh(
        core_axis_name="c", subcore_axis_name="s", num_cores=2, num_subcores=16
    )

    @functools.partial(
        pl.kernel,
        out_type=jax.ShapeDtypeStruct((rows, d_model), jnp.float32),
        mesh=mesh,
        scratch_types=[
            pltpu.VMEM((2, CH, d_model), jnp.float32),
            pltpu.VMEM((3, CH, d_model), jnp.float32),
            pltpu.SemaphoreType.DMA((2,)),
            pltpu.SemaphoreType.DMA((3,)),
            pltpu.SemaphoreType.DMA((3,)),
        ],
    )
    def k(x_hbm, pos_hbm, out_hbm, pbufs, xbufs, psem, lsem, ssem):
        wid = lax.axis_index("s") * 2 + lax.axis_index("c")
        seq0 = wid * seq_per_w
        n_steps = n_chunks * batch

        def x_row(i):
            c = i // batch
            b = i % batch
            return b * seq_len + seq0 + c * CH

        def load_copy(i, slot):
            return pltpu.make_async_copy(
                x_hbm.at[pl.ds(x_row(i), CH)], xbufs.at[slot], lsem.at[slot]
            )

        def pload_copy(c, slot):
            return pltpu.make_async_copy(
                pos_hbm.at[pl.ds(seq0 + c * CH, CH)], pbufs.at[slot], psem.at[slot]
            )

        def store_copy(i, slot):
            return pltpu.make_async_copy(
                xbufs.at[slot], out_hbm.at[pl.ds(x_row(i), CH)], ssem.at[slot]
            )

        groups = d_model // (16 * UN)

        def add(pslot, xslot):
            def add_body(f, _):
                row = f // groups
                g = f % groups
                for u in range(UN):
                    off = g * (16 * UN) + u * 16
                    xbufs[xslot, row, pl.ds(off, 16)] = (
                        xbufs[xslot, row, pl.ds(off, 16)]
                        + pbufs[pslot, row, pl.ds(off, 16)]
                    )
                return ()

            lax.fori_loop(0, CH * groups, add_body, ())

        pload_copy(0, 0).start()
        load_copy(0, 0).start()
        load_copy(1, 1).start()

        def step(j, _):
            c = j // batch
            b = j % batch
            pslot = c % 2
            xslot = j % 3

            @pl.when(b == 0)
            def _():
                pload_copy(c, pslot).wait()

                @pl.when(c + 1 < n_chunks)
                def _():
                    pload_copy(c + 1, (c + 1) % 2).start()

            load_copy(j, xslot).wait()
            add(pslot, xslot)
            store_copy(j, xslot).start()

            t = j + 2

            @pl.when(t < n_steps)
            def _():
                @pl.when(j >= 1)
                def _():
                    store_copy(j - 1, (j - 1) % 3).wait()

                load_copy(t, t % 3).start()

            return ()

        lax.fori_loop(0, n_steps, step, ())
        store_copy(n_steps - 3, (n_steps - 3) % 3).wait()
        store_copy(n_steps - 2, (n_steps - 2) % 3).wait()
        store_copy(n_steps - 1, (n_steps - 1) % 3).wait()

    return k


def kernel(x, pos_table):
    batch, seq_len, d_model = x.shape
    x2 = x.reshape(batch * seq_len, d_model)
    out = _make_sc(batch, seq_len, d_model)(x2, pos_table)
    return out.reshape(batch, seq_len, d_model)
